# baseline (speedup 1.0000x reference)
import functools

import jax
import jax.numpy as jnp
from jax import lax
from jax.experimental import pallas as pl
from jax.experimental.pallas import tpu as pltpu
from jax.experimental.pallas import tpu_sc as plsc

_B = 4
_S = 8192
_D = 1024

_info = plsc.get_sparse_core_info()
_NC = _info.num_cores
_NS = _info.num_subcores
_NW = _NC * _NS
_ROWS_PER_W = _S // _NW
_R = 8
_JOBS = _ROWS_PER_W // _R
_DEPTH = 3

_mesh = plsc.VectorSubcoreMesh(core_axis_name="c", subcore_axis_name="s")


@functools.partial(
    pl.kernel,
    out_type=jax.ShapeDtypeStruct((_B, _S, _D), jnp.float32),
    mesh=_mesh,
    scratch_types=[
        pltpu.VMEM((_R, _D), jnp.float32),
        pltpu.VMEM((_R, _D), jnp.float32),
        pltpu.VMEM((_R, _D), jnp.float32),
        pltpu.VMEM((_B * _R, _D), jnp.float32),
        pltpu.VMEM((_B * _R, _D), jnp.float32),
        pltpu.VMEM((_B * _R, _D), jnp.float32),
        pltpu.SemaphoreType.DMA,
        pltpu.SemaphoreType.DMA,
        pltpu.SemaphoreType.DMA,
        pltpu.SemaphoreType.DMA,
        pltpu.SemaphoreType.DMA,
        pltpu.SemaphoreType.DMA,
    ],
)
def _pos_add(x_hbm, pos_hbm, out_hbm, pos_v0, pos_v1, pos_v2,
             v0, v1, v2, sl0, sl1, sl2, ss0, ss1, ss2):
    wid = lax.axis_index("s") * _NC + lax.axis_index("c")
    base = wid * _ROWS_PER_W
    pos_v = (pos_v0, pos_v1, pos_v2)
    buf = (v0, v1, v2)
    sem_l = (sl0, sl1, sl2)
    sem_s = (ss0, ss1, ss2)

    def issue_loads(k, c):
        row0 = base + k * _R
        pltpu.async_copy(pos_hbm.at[pl.ds(row0, _R), :], pos_v[c], sem_l[c])
        for b in range(_B):
            pltpu.async_copy(
                x_hbm.at[b, pl.ds(row0, _R), :],
                buf[c].at[pl.ds(b * _R, _R), :],
                sem_l[c],
            )

    def wait_loads(c):
        pltpu.make_async_copy(
            pos_hbm.at[pl.ds(0, _R), :], pos_v[c], sem_l[c]).wait()
        pltpu.make_async_copy(
            x_hbm.at[0, pl.ds(0, _B * _R), :], buf[c], sem_l[c]).wait()

    def issue_stores(k, c):
        row0 = base + k * _R
        for b in range(_B):
            pltpu.async_copy(
                buf[c].at[pl.ds(b * _R, _R), :],
                out_hbm.at[b, pl.ds(row0, _R), :],
                sem_s[c],
            )

    def wait_stores(c):
        pltpu.make_async_copy(
            buf[c], out_hbm.at[0, pl.ds(0, _B * _R), :], sem_s[c]).wait()

    def compute(c):
        bc = buf[c]
        pc = pos_v[c]

        @plsc.parallel_loop(0, _D, step=16, unroll=2)
        def add_body(i):
            for r in range(_R):
                p = pc[r, pl.ds(i, 16)]
                for b in range(_B):
                    row = b * _R + r
                    bc[row, pl.ds(i, 16)] = bc[row, pl.ds(i, 16)] + p

    for c in range(_DEPTH - 1):
        issue_loads(c, c)

    def iter_body(k, _):
        c = lax.rem(k, _DEPTH)

        def run(c):
            wait_loads(c)
            compute(c)
            issue_stores(k, c)
            cp = (c + _DEPTH - 1) % _DEPTH
            @pl.when(k + _DEPTH - 1 <= _JOBS - 1)
            def _():
                @pl.when(k >= 1)
                def _():
                    wait_stores(cp)
                issue_loads(k + _DEPTH - 1, cp)

        @pl.when(c == 0)
        def _():
            run(0)

        @pl.when(c == 1)
        def _():
            run(1)

        @pl.when(c == 2)
        def _():
            run(2)

        return 0

    lax.fori_loop(0, _JOBS, iter_body, 0)
    for c in range(_DEPTH):
        wait_stores(c)


def kernel(x, position_matrix):
    return _pos_add(x, position_matrix)
